# baseline (speedup 1.0000x reference)
import functools

import jax
import jax.numpy as jnp
from jax import lax
from jax.experimental import pallas as pl
from jax.experimental.pallas import tpu as pltpu
from jax.experimental.pallas import tpu_sc as plsc

CAPACITY_FACTOR = 1.25

_NC = 2
_NS = 16
_NW = _NC * _NS

_TB = 512
_FB = 512
_CHUNK = 64


def _router_body(E, C, garbage_row,
                 x_ref, wrt_ref, br_ref,
                 slot_ref, mask_ref, pmax_ref, counts_ref):
    i = pl.program_id(0)

    @pl.when(i == 0)
    def _():
        counts_ref[...] = jnp.zeros_like(counts_ref)

    x = x_ref[...]
    logits = jnp.dot(x, wrt_ref[...],
                     preferred_element_type=jnp.float32,
                     precision=lax.Precision.DEFAULT) + br_ref[...]
    m = jnp.max(logits, axis=-1, keepdims=True)
    lane = lax.broadcasted_iota(jnp.int32, logits.shape, 1)
    route = jnp.min(jnp.where(logits >= m, lane, E), axis=-1, keepdims=True)
    pmax = 1.0 / jnp.sum(jnp.exp(logits - m), axis=-1, keepdims=True)

    onehot = (lane == route).astype(jnp.float32)
    tb = x.shape[0]
    r = lax.broadcasted_iota(jnp.int32, (tb, tb), 0)
    c = lax.broadcasted_iota(jnp.int32, (tb, tb), 1)
    tril = (r >= c).astype(jnp.float32)
    pos = jnp.dot(tril, onehot, preferred_element_type=jnp.float32)
    pos = pos + counts_ref[...]
    counts_ref[...] = pos[-1:, :]
    posi = jnp.sum(pos * onehot, axis=-1, keepdims=True).astype(jnp.int32) - 1
    keep = posi < C
    slot = jnp.where(keep, route * C + posi, garbage_row)
    slot_ref[...] = slot
    mask_ref[...] = keep.astype(jnp.float32)
    pmax_ref[...] = pmax


def _route_tokens(xf, Wr, br):
    T, D = xf.shape
    E = Wr.shape[0]
    C = int(CAPACITY_FACTOR * T / E)
    ec_pad = E * C + 8
    nb = T // _TB
    out_shapes = (
        jax.ShapeDtypeStruct((T, 1), jnp.int32),
        jax.ShapeDtypeStruct((T, 1), jnp.float32),
        jax.ShapeDtypeStruct((T, 1), jnp.float32),
    )
    slot, mask, pmax = pl.pallas_call(
        functools.partial(_router_body, E, C, E * C),
        grid=(nb,),
        in_specs=[
            pl.BlockSpec((_TB, D), lambda i: (i, 0)),
            pl.BlockSpec((D, E), lambda i: (0, 0)),
            pl.BlockSpec((1, E), lambda i: (0, 0)),
        ],
        out_specs=(
            pl.BlockSpec((_TB, 1), lambda i: (i, 0)),
            pl.BlockSpec((_TB, 1), lambda i: (i, 0)),
            pl.BlockSpec((_TB, 1), lambda i: (i, 0)),
        ),
        out_shape=out_shapes,
        scratch_shapes=[pltpu.VMEM((1, E), jnp.float32)],
    )(xf, Wr.T, br.reshape(1, E))
    return slot.reshape(T), mask, pmax, C, ec_pad


def _sc_scatter(xf, slot, ec_pad):
    T, D = xf.shape
    per_w = T // _NW
    mesh = plsc.VectorSubcoreMesh(core_axis_name="c", subcore_axis_name="s")

    @functools.partial(
        pl.kernel,
        mesh=mesh,
        out_type=jax.ShapeDtypeStruct((ec_pad, D), jnp.float32),
        scratch_types=[
            pltpu.VMEM((_CHUNK,), jnp.int32),
            pltpu.VMEM((_CHUNK, D), jnp.float32),
            pltpu.SemaphoreType.DMA,
        ],
    )
    def scatter_k(x_hbm, idx_hbm, out_hbm, idx_v, rows_v, sem):
        wid = lax.axis_index("s") * _NC + lax.axis_index("c")
        for j in range(per_w // _CHUNK):
            base = wid * per_w + j * _CHUNK
            pltpu.sync_copy(idx_hbm.at[pl.ds(base, _CHUNK)], idx_v)
            pltpu.sync_copy(x_hbm.at[pl.ds(base, _CHUNK)], rows_v)
            pltpu.async_copy(rows_v, out_hbm.at[idx_v], sem).wait()

    return scatter_k(xf, slot)


def _sc_gather(ybuf, slot, T):
    D = ybuf.shape[1]
    per_w = T // _NW
    mesh = plsc.VectorSubcoreMesh(core_axis_name="c", subcore_axis_name="s")

    @functools.partial(
        pl.kernel,
        mesh=mesh,
        out_type=jax.ShapeDtypeStruct((T, D), jnp.float32),
        scratch_types=[
            pltpu.VMEM((_CHUNK,), jnp.int32),
            pltpu.VMEM((_CHUNK, D), jnp.float32),
            pltpu.SemaphoreType.DMA,
        ],
    )
    def gather_k(y_hbm, idx_hbm, out_hbm, idx_v, rows_v, sem):
        wid = lax.axis_index("s") * _NC + lax.axis_index("c")
        for j in range(per_w // _CHUNK):
            base = wid * per_w + j * _CHUNK
            pltpu.sync_copy(idx_hbm.at[pl.ds(base, _CHUNK)], idx_v)
            pltpu.async_copy(y_hbm.at[idx_v], rows_v, sem).wait()
            pltpu.sync_copy(rows_v, out_hbm.at[pl.ds(base, _CHUNK)])

    return gather_k(ybuf, slot)


def _ffn_body(nfb, x_ref, w1_ref, b1_ref, w2_ref, b2_ref, out_ref, acc_ref):
    fb = pl.program_id(1)

    @pl.when(fb == 0)
    def _():
        acc_ref[...] = jnp.zeros_like(acc_ref)

    h = jnp.dot(x_ref[...], w1_ref[0],
                preferred_element_type=jnp.float32,
                precision=lax.Precision.HIGHEST) + b1_ref[0]
    h = jnp.maximum(h, 0.0)
    acc_ref[...] += jnp.dot(h, w2_ref[0],
                            preferred_element_type=jnp.float32,
                            precision=lax.Precision.HIGHEST)

    @pl.when(fb == nfb - 1)
    def _():
        out_ref[...] = acc_ref[...] + b2_ref[0]


def _expert_ffn(buf, W1, b1, W2, b2, C):
    ec_pad, D = buf.shape
    E, _, F = W1.shape
    nfb = F // _FB
    return pl.pallas_call(
        functools.partial(_ffn_body, nfb),
        grid=(E, nfb),
        in_specs=[
            pl.BlockSpec((C, D), lambda e, f: (e, 0)),
            pl.BlockSpec((1, D, _FB), lambda e, f: (e, 0, f)),
            pl.BlockSpec((1, 1, _FB), lambda e, f: (e, 0, f)),
            pl.BlockSpec((1, _FB, D), lambda e, f: (e, f, 0)),
            pl.BlockSpec((1, 1, D), lambda e, f: (e, 0, 0)),
        ],
        out_specs=pl.BlockSpec((C, D), lambda e, f: (e, 0)),
        out_shape=jax.ShapeDtypeStruct((ec_pad, D), jnp.float32),
        scratch_shapes=[pltpu.VMEM((C, D), jnp.float32)],
    )(buf, W1, b1.reshape(E, 1, F), W2, b2.reshape(E, 1, D))


def _combine_body(g_ref, x_ref, mask_ref, pmax_ref, out_ref):
    kept = mask_ref[...] > 0.5
    out_ref[...] = jnp.where(kept, g_ref[...], x_ref[...]) * pmax_ref[...]


def _combine(gathered, xf, mask, pmax):
    T, D = xf.shape
    nb = T // _TB
    return pl.pallas_call(
        _combine_body,
        grid=(nb,),
        in_specs=[
            pl.BlockSpec((_TB, D), lambda i: (i, 0)),
            pl.BlockSpec((_TB, D), lambda i: (i, 0)),
            pl.BlockSpec((_TB, 1), lambda i: (i, 0)),
            pl.BlockSpec((_TB, 1), lambda i: (i, 0)),
        ],
        out_specs=pl.BlockSpec((_TB, D), lambda i: (i, 0)),
        out_shape=jax.ShapeDtypeStruct((T, D), jnp.float32),
    )(gathered, xf, mask, pmax)


def kernel(x, Wr, br, W1, b1, W2, b2):
    seq, bsz, D = x.shape
    T = seq * bsz
    xf = x.reshape(T, D)
    slot, mask, pmax, C, ec_pad = _route_tokens(xf, Wr, br)
    buf = _sc_scatter(xf, slot, ec_pad)
    ybuf = _expert_ffn(buf, W1, b1, W2, b2, C)
    gathered = _sc_gather(ybuf, slot, T)
    final = _combine(gathered, xf, mask, pmax)
    return final.reshape(seq, bsz, D)


# speedup vs baseline: 2.0179x; 2.0179x over previous
import functools

import jax
import jax.numpy as jnp
from jax import lax
from jax.experimental import pallas as pl
from jax.experimental.pallas import tpu as pltpu
from jax.experimental.pallas import tpu_sc as plsc

CAPACITY_FACTOR = 1.25

_NC = 2
_NS = 16
_NW = _NC * _NS

_TB = 512
_FB = 512
_CHUNK = 64


def _router_body(E, C, garbage_row,
                 x_ref, wrt_ref, br_ref,
                 slot_ref, mask_ref, pmax_ref, counts_ref):
    i = pl.program_id(0)

    @pl.when(i == 0)
    def _():
        counts_ref[...] = jnp.zeros_like(counts_ref)

    x = x_ref[...]
    logits = jnp.dot(x, wrt_ref[...],
                     preferred_element_type=jnp.float32,
                     precision=lax.Precision.DEFAULT) + br_ref[...]
    m = jnp.max(logits, axis=-1, keepdims=True)
    lane = lax.broadcasted_iota(jnp.int32, logits.shape, 1)
    route = jnp.min(jnp.where(logits >= m, lane, E), axis=-1, keepdims=True)
    pmax = 1.0 / jnp.sum(jnp.exp(logits - m), axis=-1, keepdims=True)

    onehot = (lane == route).astype(jnp.float32)
    tb = x.shape[0]
    r = lax.broadcasted_iota(jnp.int32, (tb, tb), 0)
    c = lax.broadcasted_iota(jnp.int32, (tb, tb), 1)
    tril = (r >= c).astype(jnp.float32)
    pos = jnp.dot(tril, onehot, preferred_element_type=jnp.float32)
    pos = pos + counts_ref[...]
    counts_ref[...] = pos[-1:, :]
    posi = jnp.sum(pos * onehot, axis=-1, keepdims=True).astype(jnp.int32) - 1
    keep = posi < C
    slot = jnp.where(keep, route * C + posi, garbage_row)
    slot_ref[...] = slot
    mask_ref[...] = keep.astype(jnp.float32)
    pmax_ref[...] = pmax


def _route_tokens(xf, Wr, br):
    T, D = xf.shape
    E = Wr.shape[0]
    C = int(CAPACITY_FACTOR * T / E)
    ec_pad = E * C + 8
    nb = T // _TB
    out_shapes = (
        jax.ShapeDtypeStruct((T, 1), jnp.int32),
        jax.ShapeDtypeStruct((T, 1), jnp.float32),
        jax.ShapeDtypeStruct((T, 1), jnp.float32),
    )
    slot, mask, pmax = pl.pallas_call(
        functools.partial(_router_body, E, C, E * C),
        grid=(nb,),
        in_specs=[
            pl.BlockSpec((_TB, D), lambda i: (i, 0)),
            pl.BlockSpec((D, E), lambda i: (0, 0)),
            pl.BlockSpec((1, E), lambda i: (0, 0)),
        ],
        out_specs=(
            pl.BlockSpec((_TB, 1), lambda i: (i, 0)),
            pl.BlockSpec((_TB, 1), lambda i: (i, 0)),
            pl.BlockSpec((_TB, 1), lambda i: (i, 0)),
        ),
        out_shape=out_shapes,
        scratch_shapes=[pltpu.VMEM((1, E), jnp.float32)],
    )(xf, Wr.T, br.reshape(1, E))
    return slot.reshape(T), mask, pmax, C, ec_pad


def _sc_scatter(xf, slot, ec_pad):
    T, D = xf.shape
    per_w = T // _NW
    mesh = plsc.VectorSubcoreMesh(core_axis_name="c", subcore_axis_name="s")

    @functools.partial(
        pl.kernel,
        mesh=mesh,
        out_type=jax.ShapeDtypeStruct((ec_pad, D), jnp.float32),
        scratch_types=[
            pltpu.VMEM((_CHUNK,), jnp.int32),
            pltpu.VMEM((_CHUNK, D), jnp.float32),
            pltpu.SemaphoreType.DMA,
        ],
    )
    def scatter_k(x_hbm, idx_hbm, out_hbm, idx_v, rows_v, sem):
        wid = lax.axis_index("s") * _NC + lax.axis_index("c")
        for j in range(per_w // _CHUNK):
            base = wid * per_w + j * _CHUNK
            pltpu.sync_copy(idx_hbm.at[pl.ds(base, _CHUNK)], idx_v)
            pltpu.sync_copy(x_hbm.at[pl.ds(base, _CHUNK)], rows_v)
            pltpu.async_copy(rows_v, out_hbm.at[idx_v], sem).wait()

    return scatter_k(xf, slot)


def _sc_gather(ybuf, slot, T):
    D = ybuf.shape[1]
    per_w = T // _NW
    mesh = plsc.VectorSubcoreMesh(core_axis_name="c", subcore_axis_name="s")

    @functools.partial(
        pl.kernel,
        mesh=mesh,
        out_type=jax.ShapeDtypeStruct((T, D), jnp.float32),
        scratch_types=[
            pltpu.VMEM((_CHUNK,), jnp.int32),
            pltpu.VMEM((_CHUNK, D), jnp.float32),
            pltpu.SemaphoreType.DMA,
        ],
    )
    def gather_k(y_hbm, idx_hbm, out_hbm, idx_v, rows_v, sem):
        wid = lax.axis_index("s") * _NC + lax.axis_index("c")
        for j in range(per_w // _CHUNK):
            base = wid * per_w + j * _CHUNK
            pltpu.sync_copy(idx_hbm.at[pl.ds(base, _CHUNK)], idx_v)
            pltpu.async_copy(y_hbm.at[idx_v], rows_v, sem).wait()
            pltpu.sync_copy(rows_v, out_hbm.at[pl.ds(base, _CHUNK)])

    return gather_k(ybuf, slot)


def _ffn_body(nfb, x_ref, w1_ref, b1_ref, w2_ref, b2_ref, out_ref, acc_ref):
    fb = pl.program_id(1)

    @pl.when(fb == 0)
    def _():
        acc_ref[...] = jnp.zeros_like(acc_ref)

    h = jnp.dot(x_ref[...], w1_ref[0],
                preferred_element_type=jnp.float32,
                precision=lax.Precision.DEFAULT) + b1_ref[0]
    h = jnp.maximum(h, 0.0)
    acc_ref[...] += jnp.dot(h, w2_ref[0],
                            preferred_element_type=jnp.float32,
                            precision=lax.Precision.DEFAULT)

    @pl.when(fb == nfb - 1)
    def _():
        out_ref[...] = acc_ref[...] + b2_ref[0]


def _expert_ffn(buf, W1, b1, W2, b2, C):
    ec_pad, D = buf.shape
    E, _, F = W1.shape
    nfb = F // _FB
    return pl.pallas_call(
        functools.partial(_ffn_body, nfb),
        grid=(E, nfb),
        in_specs=[
            pl.BlockSpec((C, D), lambda e, f: (e, 0)),
            pl.BlockSpec((1, D, _FB), lambda e, f: (e, 0, f)),
            pl.BlockSpec((1, 1, _FB), lambda e, f: (e, 0, f)),
            pl.BlockSpec((1, _FB, D), lambda e, f: (e, f, 0)),
            pl.BlockSpec((1, 1, D), lambda e, f: (e, 0, 0)),
        ],
        out_specs=pl.BlockSpec((C, D), lambda e, f: (e, 0)),
        out_shape=jax.ShapeDtypeStruct((ec_pad, D), jnp.float32),
        scratch_shapes=[pltpu.VMEM((C, D), jnp.float32)],
    )(buf, W1, b1.reshape(E, 1, F), W2, b2.reshape(E, 1, D))


def _combine_body(g_ref, x_ref, mask_ref, pmax_ref, out_ref):
    kept = mask_ref[...] > 0.5
    out_ref[...] = jnp.where(kept, g_ref[...], x_ref[...]) * pmax_ref[...]


def _combine(gathered, xf, mask, pmax):
    T, D = xf.shape
    nb = T // _TB
    return pl.pallas_call(
        _combine_body,
        grid=(nb,),
        in_specs=[
            pl.BlockSpec((_TB, D), lambda i: (i, 0)),
            pl.BlockSpec((_TB, D), lambda i: (i, 0)),
            pl.BlockSpec((_TB, 1), lambda i: (i, 0)),
            pl.BlockSpec((_TB, 1), lambda i: (i, 0)),
        ],
        out_specs=pl.BlockSpec((_TB, D), lambda i: (i, 0)),
        out_shape=jax.ShapeDtypeStruct((T, D), jnp.float32),
    )(gathered, xf, mask, pmax)


def kernel(x, Wr, br, W1, b1, W2, b2):
    seq, bsz, D = x.shape
    T = seq * bsz
    xf = x.reshape(T, D)
    slot, mask, pmax, C, ec_pad = _route_tokens(xf, Wr, br)
    buf = _sc_scatter(xf, slot, ec_pad)
    ybuf = _expert_ffn(buf, W1, b1, W2, b2, C)
    gathered = _sc_gather(ybuf, slot, T)
    final = _combine(gathered, xf, mask, pmax)
    return final.reshape(seq, bsz, D)


# speedup vs baseline: 2.1904x; 1.0855x over previous
import functools

import jax
import jax.numpy as jnp
from jax import lax
from jax.experimental import pallas as pl
from jax.experimental.pallas import tpu as pltpu
from jax.experimental.pallas import tpu_sc as plsc

CAPACITY_FACTOR = 1.25

_NC = 2
_NS = 16
_NW = _NC * _NS

_TB = 512
_CHUNK = 32
_LANES = 16


def _router_body(E, C, ec_pad,
                 x_ref, wrt_ref, br_ref,
                 slot_ref, pmax_ref, counts_ref):
    i = pl.program_id(0)

    @pl.when(i == 0)
    def _():
        counts_ref[...] = jnp.zeros_like(counts_ref)

    x = x_ref[...]
    logits = jnp.dot(x, wrt_ref[...],
                     preferred_element_type=jnp.float32,
                     precision=lax.Precision.DEFAULT) + br_ref[...]
    m = jnp.max(logits, axis=-1, keepdims=True)
    lane = lax.broadcasted_iota(jnp.int32, logits.shape, 1)
    route = jnp.min(jnp.where(logits >= m, lane, E), axis=-1, keepdims=True)
    pmax = 1.0 / jnp.sum(jnp.exp(logits - m), axis=-1, keepdims=True)

    onehot = (lane == route).astype(jnp.float32)
    tb = x.shape[0]
    r = lax.broadcasted_iota(jnp.int32, (tb, tb), 0)
    c = lax.broadcasted_iota(jnp.int32, (tb, tb), 1)
    tril = (r >= c).astype(jnp.float32)
    pos = jnp.dot(tril, onehot, preferred_element_type=jnp.float32)
    pos = pos + counts_ref[...]
    counts_ref[...] = pos[-1:, :]
    posi = jnp.sum(pos * onehot, axis=-1, keepdims=True).astype(jnp.int32) - 1
    keep = posi < C
    gidx = i * tb + lax.broadcasted_iota(jnp.int32, (tb, 1), 0)
    slot_ref[...] = jnp.where(keep, route * C + posi, ec_pad + gidx)
    pmax_ref[...] = jnp.broadcast_to(pmax, (tb, _LANES))


def _route_tokens(xf, Wr, br, C, ec_pad):
    T, D = xf.shape
    E = Wr.shape[0]
    nb = T // _TB
    out_shapes = (
        jax.ShapeDtypeStruct((T, 1), jnp.int32),
        jax.ShapeDtypeStruct((T, _LANES), jnp.float32),
    )
    slot, pmax = pl.pallas_call(
        functools.partial(_router_body, E, C, ec_pad),
        grid=(nb,),
        in_specs=[
            pl.BlockSpec((_TB, D), lambda i: (i, 0)),
            pl.BlockSpec((D, E), lambda i: (0, 0)),
            pl.BlockSpec((1, E), lambda i: (0, 0)),
        ],
        out_specs=(
            pl.BlockSpec((_TB, 1), lambda i: (i, 0)),
            pl.BlockSpec((_TB, _LANES), lambda i: (i, 0)),
        ),
        out_shape=out_shapes,
        scratch_shapes=[pltpu.VMEM((1, E), jnp.float32)],
    )(xf, Wr.T, br.reshape(1, E))
    return slot.reshape(T), pmax


def _sc_scatter(xf, slot, nrows):
    T, D = xf.shape
    per_w = T // _NW
    mesh = plsc.VectorSubcoreMesh(core_axis_name="c", subcore_axis_name="s")

    @functools.partial(
        pl.kernel,
        mesh=mesh,
        out_type=jax.ShapeDtypeStruct((nrows, D), jnp.float32),
        scratch_types=[
            pltpu.VMEM((_CHUNK,), jnp.int32),
            pltpu.VMEM((_CHUNK, D), jnp.float32),
            pltpu.SemaphoreType.DMA,
        ],
    )
    def scatter_k(x_hbm, idx_hbm, out_hbm, idx_v, rows_v, sem):
        wid = lax.axis_index("s") * _NC + lax.axis_index("c")
        for j in range(per_w // _CHUNK):
            base = wid * per_w + j * _CHUNK
            pltpu.sync_copy(idx_hbm.at[pl.ds(base, _CHUNK)], idx_v)
            pltpu.sync_copy(x_hbm.at[pl.ds(base, _CHUNK)], rows_v)
            pltpu.async_copy(rows_v, out_hbm.at[idx_v], sem).wait()

    return scatter_k(xf, slot)


def _sc_gather_scale(ybuf, slot, pmax, T):
    D = ybuf.shape[1]
    per_w = T // _NW
    groups = D // _LANES
    mesh = plsc.VectorSubcoreMesh(core_axis_name="c", subcore_axis_name="s")

    @functools.partial(
        pl.kernel,
        mesh=mesh,
        out_type=jax.ShapeDtypeStruct((T, D), jnp.float32),
        scratch_types=[
            pltpu.VMEM((_CHUNK,), jnp.int32),
            pltpu.VMEM((_CHUNK, _LANES), jnp.float32),
            pltpu.VMEM((_CHUNK, D), jnp.float32),
            pltpu.SemaphoreType.DMA,
        ],
    )
    def gather_k(y_hbm, idx_hbm, pmax_hbm, out_hbm, idx_v, pmax_v, rows_v, sem):
        wid = lax.axis_index("s") * _NC + lax.axis_index("c")
        for j in range(per_w // _CHUNK):
            base = wid * per_w + j * _CHUNK
            pltpu.sync_copy(idx_hbm.at[pl.ds(base, _CHUNK)], idx_v)
            pltpu.sync_copy(pmax_hbm.at[pl.ds(base, _CHUNK)], pmax_v)
            pltpu.async_copy(y_hbm.at[idx_v], rows_v, sem).wait()
            for r in range(_CHUNK):
                pv = pmax_v[r]

                @pl.loop(0, groups, unroll=8)
                def _(g, r=r, pv=pv):
                    sl = pl.ds(g * _LANES, _LANES)
                    rows_v[r, sl] = rows_v[r, sl] * pv
            pltpu.sync_copy(rows_v, out_hbm.at[pl.ds(base, _CHUNK)])

    return gather_k(ybuf, slot, pmax)


def _ffn_body(x_ref, w1_ref, b1_ref, w2_ref, b2_ref, out_ref):
    h = jnp.dot(x_ref[...], w1_ref[0],
                preferred_element_type=jnp.float32,
                precision=lax.Precision.DEFAULT) + b1_ref[0]
    h = jnp.maximum(h, 0.0)
    out_ref[...] = jnp.dot(h, w2_ref[0],
                           preferred_element_type=jnp.float32,
                           precision=lax.Precision.DEFAULT) + b2_ref[0]


def _expert_ffn(buf, W1, b1, W2, b2, C):
    nrows, D = buf.shape
    E, _, F = W1.shape
    return pl.pallas_call(
        _ffn_body,
        grid=(E,),
        in_specs=[
            pl.BlockSpec((C, D), lambda e: (e, 0)),
            pl.BlockSpec((1, D, F), lambda e: (e, 0, 0)),
            pl.BlockSpec((1, 1, F), lambda e: (e, 0, 0)),
            pl.BlockSpec((1, F, D), lambda e: (e, 0, 0)),
            pl.BlockSpec((1, 1, D), lambda e: (e, 0, 0)),
        ],
        out_specs=pl.BlockSpec((C, D), lambda e: (e, 0)),
        out_shape=jax.ShapeDtypeStruct((nrows, D), jnp.float32),
        input_output_aliases={0: 0},
    )(buf, W1, b1.reshape(E, 1, F), W2, b2.reshape(E, 1, D))


def kernel(x, Wr, br, W1, b1, W2, b2):
    seq, bsz, D = x.shape
    T = seq * bsz
    E = Wr.shape[0]
    C = int(CAPACITY_FACTOR * T / E)
    ec_pad = E * C + 8
    nrows = ec_pad + T
    xf = x.reshape(T, D)
    slot, pmax = _route_tokens(xf, Wr, br, C, ec_pad)
    buf = _sc_scatter(xf, slot, nrows)
    ybuf = _expert_ffn(buf, W1, b1, W2, b2, C)
    final = _sc_gather_scale(ybuf, slot, pmax, T)
    return final.reshape(seq, bsz, D)


# speedup vs baseline: 2.3052x; 1.0524x over previous
import functools

import jax
import jax.numpy as jnp
from jax import lax
from jax.experimental import pallas as pl
from jax.experimental.pallas import tpu as pltpu
from jax.experimental.pallas import tpu_sc as plsc

CAPACITY_FACTOR = 1.25

_NC = 2
_NS = 16
_NW = _NC * _NS

_TB = 512
_CHUNK = 16
_RING = 4
_LANES = 16
_SCW = 128


def _router_body(E, C, ec_pad,
                 x_ref, wrt_ref, br_ref,
                 slot_ref, pmax_ref, scale_ref, counts_ref):
    i = pl.program_id(0)

    @pl.when(i == 0)
    def _():
        counts_ref[...] = jnp.zeros_like(counts_ref)

    x = x_ref[...]
    logits = jnp.dot(x, wrt_ref[...],
                     preferred_element_type=jnp.float32,
                     precision=lax.Precision.DEFAULT) + br_ref[...]
    m = jnp.max(logits, axis=-1, keepdims=True)
    lane = lax.broadcasted_iota(jnp.int32, logits.shape, 1)
    route = jnp.min(jnp.where(logits >= m, lane, E), axis=-1, keepdims=True)
    pmax = 1.0 / jnp.sum(jnp.exp(logits - m), axis=-1, keepdims=True)

    onehot = (lane == route).astype(jnp.float32)
    tb = x.shape[0]
    r = lax.broadcasted_iota(jnp.int32, (tb, tb), 0)
    c = lax.broadcasted_iota(jnp.int32, (tb, tb), 1)
    tril = (r >= c).astype(jnp.float32)
    pos = jnp.dot(tril, onehot, preferred_element_type=jnp.float32)
    pos = pos + counts_ref[...]
    counts_ref[...] = pos[-1:, :]
    posi = jnp.sum(pos * onehot, axis=-1, keepdims=True).astype(jnp.int32) - 1
    keep = posi < C
    gidx = i * tb + lax.broadcasted_iota(jnp.int32, (tb, 1), 0)
    slot_ref[...] = jnp.where(keep, route * C + posi, ec_pad + gidx)
    pmax_ref[...] = jnp.broadcast_to(pmax, (tb, _SCW))
    scale_ref[...] = jnp.broadcast_to(
        jnp.where(keep, 1.0, pmax), (tb, _LANES))


def _route_tokens(xf, Wr, br, C, ec_pad):
    T, D = xf.shape
    E = Wr.shape[0]
    nb = T // _TB
    out_shapes = (
        jax.ShapeDtypeStruct((T, 1), jnp.int32),
        jax.ShapeDtypeStruct((T, _SCW), jnp.float32),
        jax.ShapeDtypeStruct((T, _LANES), jnp.float32),
    )
    slot, pmax, scale = pl.pallas_call(
        functools.partial(_router_body, E, C, ec_pad),
        grid=(nb,),
        in_specs=[
            pl.BlockSpec((_TB, D), lambda i: (i, 0)),
            pl.BlockSpec((D, E), lambda i: (0, 0)),
            pl.BlockSpec((1, E), lambda i: (0, 0)),
        ],
        out_specs=(
            pl.BlockSpec((_TB, 1), lambda i: (i, 0)),
            pl.BlockSpec((_TB, _SCW), lambda i: (i, 0)),
            pl.BlockSpec((_TB, _LANES), lambda i: (i, 0)),
        ),
        out_shape=out_shapes,
        scratch_shapes=[pltpu.VMEM((1, E), jnp.float32)],
    )(xf, Wr.T, br.reshape(1, E))
    return slot.reshape(T), pmax, scale


def _sc_scatter(xf, slot, pmax, nrows):
    T, D = xf.shape
    per_w = T // _NW
    nj = per_w // _CHUNK
    mesh = plsc.VectorSubcoreMesh(core_axis_name="c", subcore_axis_name="s")

    scratch = (
        [pltpu.VMEM((_CHUNK,), jnp.int32) for _ in range(_RING)]
        + [pltpu.VMEM((_CHUNK, D), jnp.float32) for _ in range(_RING)]
        + [pltpu.VMEM((_CHUNK, _SCW), jnp.float32) for _ in range(_RING)]
        + [pltpu.SemaphoreType.DMA for _ in range(2 * _RING)]
    )

    @functools.partial(
        pl.kernel,
        mesh=mesh,
        out_type=(
            jax.ShapeDtypeStruct((nrows, D), jnp.float32),
            jax.ShapeDtypeStruct((nrows, _SCW), jnp.float32),
        ),
        scratch_types=scratch,
    )
    def scatter_k(x_hbm, idx_hbm, pm_hbm, buf_out, scale_out, *sc):
        idx_v = sc[0:_RING]
        rows_v = sc[_RING:2 * _RING]
        pm_v = sc[2 * _RING:3 * _RING]
        lsem = sc[3 * _RING:4 * _RING]
        ssem = sc[4 * _RING:5 * _RING]
        wid = lax.axis_index("s") * _NC + lax.axis_index("c")
        base0 = wid * per_w

        def issue_loads(j):
            k = j % _RING
            b = base0 + j * _CHUNK
            return (
                pltpu.async_copy(idx_hbm.at[pl.ds(b, _CHUNK)], idx_v[k], lsem[k]),
                pltpu.async_copy(x_hbm.at[pl.ds(b, _CHUNK)], rows_v[k], lsem[k]),
                pltpu.async_copy(pm_hbm.at[pl.ds(b, _CHUNK)], pm_v[k], lsem[k]),
            )

        ld = [None] * nj
        st = [None] * nj
        for j in range(min(_RING - 1, nj)):
            ld[j] = issue_loads(j)
        for j in range(nj):
            k = j % _RING
            for d in ld[j]:
                d.wait()
            st[j] = (
                pltpu.async_copy(rows_v[k], buf_out.at[idx_v[k]], ssem[k]),
                pltpu.async_copy(pm_v[k], scale_out.at[idx_v[k]], ssem[k]),
            )
            nxt = j + _RING - 1
            if nxt < nj:
                if nxt >= _RING:
                    for d in st[nxt - _RING]:
                        d.wait()
                    st[nxt - _RING] = ()
                ld[nxt] = issue_loads(nxt)
        for s in st:
            for d in (s or ()):
                d.wait()

    return scatter_k(xf, slot, pmax)


def _sc_gather(ybuf, slot, scale, T):
    D = ybuf.shape[1]
    per_w = T // _NW
    nj = per_w // _CHUNK
    groups = D // _LANES
    mesh = plsc.VectorSubcoreMesh(core_axis_name="c", subcore_axis_name="s")

    scratch = (
        [pltpu.VMEM((_CHUNK,), jnp.int32) for _ in range(_RING)]
        + [pltpu.VMEM((_CHUNK, D), jnp.float32) for _ in range(_RING)]
        + [pltpu.VMEM((_CHUNK, _LANES), jnp.float32) for _ in range(_RING)]
        + [pltpu.SemaphoreType.DMA for _ in range(3 * _RING)]
    )

    @functools.partial(
        pl.kernel,
        mesh=mesh,
        out_type=jax.ShapeDtypeStruct((T, D), jnp.float32),
        scratch_types=scratch,
    )
    def gather_k(y_hbm, idx_hbm, sc_hbm, out_hbm, *sc):
        idx_v = sc[0:_RING]
        rows_v = sc[_RING:2 * _RING]
        sc_v = sc[2 * _RING:3 * _RING]
        lsem = sc[3 * _RING:4 * _RING]
        gsem = sc[4 * _RING:5 * _RING]
        ssem = sc[5 * _RING:6 * _RING]
        wid = lax.axis_index("s") * _NC + lax.axis_index("c")
        base0 = wid * per_w

        def issue_loads(j):
            k = j % _RING
            b = base0 + j * _CHUNK
            return (
                pltpu.async_copy(idx_hbm.at[pl.ds(b, _CHUNK)], idx_v[k], lsem[k]),
                pltpu.async_copy(sc_hbm.at[pl.ds(b, _CHUNK)], sc_v[k], lsem[k]),
            )

        def post_gather(j):
            k = j % _RING
            for r in range(_CHUNK):
                sv = sc_v[k][r]
                s = sv[0]

                @pl.when(s != 1.0)
                def _(k=k, r=r, sv=sv):
                    @pl.loop(0, groups, unroll=8)
                    def _(g, k=k, r=r, sv=sv):
                        sl = pl.ds(g * _LANES, _LANES)
                        rows_v[k][r, sl] = rows_v[k][r, sl] * sv

            b = base0 + j * _CHUNK
            return pltpu.async_copy(rows_v[k], out_hbm.at[pl.ds(b, _CHUNK)],
                                    ssem[k])

        ld = [None] * nj
        g = [None] * nj
        st = [None] * nj
        for j in range(min(_RING - 1, nj)):
            ld[j] = issue_loads(j)
        for j in range(nj):
            k = j % _RING
            for d in ld[j]:
                d.wait()
            g[j] = pltpu.async_copy(y_hbm.at[idx_v[k]], rows_v[k], gsem[k])
            if j > 0:
                g[j - 1].wait()
                st[j - 1] = post_gather(j - 1)
            nxt = j + _RING - 1
            if nxt < nj:
                if nxt >= _RING and st[nxt - _RING] is not None:
                    st[nxt - _RING].wait()
                    st[nxt - _RING] = False
                ld[nxt] = issue_loads(nxt)
        g[nj - 1].wait()
        st[nj - 1] = post_gather(nj - 1)
        for s in st:
            if s:
                s.wait()

    return gather_k(ybuf, slot, scale)


def _ffn_body(x_ref, w1_ref, b1_ref, w2_ref, b2_ref, s_ref, out_ref):
    h = jnp.dot(x_ref[...], w1_ref[0],
                preferred_element_type=jnp.float32,
                precision=lax.Precision.DEFAULT) + b1_ref[0]
    h = jnp.maximum(h, 0.0)
    y = jnp.dot(h, w2_ref[0],
                preferred_element_type=jnp.float32,
                precision=lax.Precision.DEFAULT) + b2_ref[0]
    out_ref[...] = y * s_ref[...][:, 0:1]


def _expert_ffn(buf, scale_buf, W1, b1, W2, b2, C):
    nrows, D = buf.shape
    E, _, F = W1.shape
    return pl.pallas_call(
        _ffn_body,
        grid=(E,),
        in_specs=[
            pl.BlockSpec((C, D), lambda e: (e, 0)),
            pl.BlockSpec((1, D, F), lambda e: (e, 0, 0)),
            pl.BlockSpec((1, 1, F), lambda e: (e, 0, 0)),
            pl.BlockSpec((1, F, D), lambda e: (e, 0, 0)),
            pl.BlockSpec((1, 1, D), lambda e: (e, 0, 0)),
            pl.BlockSpec((C, _SCW), lambda e: (e, 0)),
        ],
        out_specs=pl.BlockSpec((C, D), lambda e: (e, 0)),
        out_shape=jax.ShapeDtypeStruct((nrows, D), jnp.float32),
        input_output_aliases={0: 0},
    )(buf, W1, b1.reshape(E, 1, F), W2, b2.reshape(E, 1, D), scale_buf)


def kernel(x, Wr, br, W1, b1, W2, b2):
    seq, bsz, D = x.shape
    T = seq * bsz
    E = Wr.shape[0]
    C = int(CAPACITY_FACTOR * T / E)
    ec_pad = E * C + 8
    nrows = ec_pad + T
    xf = x.reshape(T, D)
    slot, pmax, scale = _route_tokens(xf, Wr, br, C, ec_pad)
    buf, scale_buf = _sc_scatter(xf, slot, pmax, nrows)
    ybuf = _expert_ffn(buf, scale_buf, W1, b1, W2, b2, C)
    final = _sc_gather(ybuf, slot, scale, T)
    return final.reshape(seq, bsz, D)
